# baseline (device time: 57092 ns/iter reference)
import jax
import jax.numpy as jnp
from jax import lax
from jax.experimental import pallas as pl
from jax.experimental.pallas import tpu as pltpu

N_DEV = 8
P = 4
X_ORDER = (4, 5, 3, 6, 2, 1, 7, 0)


def kernel(x, W1, W2):
    m, k = x.shape
    _, d = W1.shape
    _, f = W2.shape
    chunk = m // N_DEV
    hd = d // 2
    rh = chunk // P

    def body(x_ref, w1_ref, w2_ref, out_ref, xv, part_ref,
             rsRA, rsLA, rsRB, rsLB, own_ref, agR, agL, outv,
             x_sem, out_sem,
             rsRA_s, rsRA_r, rsLA_s, rsLA_r, rsRB_s, rsRB_r,
             rsLB_s, rsLB_r, agR_s, agR_r, agL_s, agL_r):
        i = lax.axis_index("i")
        left = lax.rem(i - 1 + N_DEV, N_DEV)
        right = lax.rem(i + 1, N_DEV)

        x_copies = {}
        for o in X_ORDER:
            c = lax.rem(i + o, N_DEV)
            cp = pltpu.make_async_copy(
                x_ref.at[pl.ds(c * chunk, chunk)], xv.at[o], x_sem.at[o])
            cp.start()
            x_copies[o] = cp

        barrier_sem = pltpu.get_barrier_semaphore()
        for nbr in (left, right):
            pl.semaphore_signal(
                barrier_sem, inc=1,
                device_id=(nbr,), device_id_type=pl.DeviceIdType.MESH,
            )
        pl.semaphore_wait(barrier_sem, 2)

        w1 = w1_ref[...].astype(jnp.bfloat16)
        w1R = w1[:, :hd]
        w1L = w1[:, hd:]
        w2R = w2_ref[:hd, :].astype(jnp.bfloat16)
        w2L = w2_ref[hd:, :].astype(jnp.bfloat16)

        xcache = {}

        def xchunk(o):
            if o not in xcache:
                x_copies[o].wait()
                xcache[o] = xv[o].astype(jnp.bfloat16)
            return xcache[o]

        def rows(p):
            return pl.ds(p * rh, rh)

        sent = []
        out_copies = []

        def store_out(slot, c, value):
            outv[slot] = value
            cp = pltpu.make_async_copy(
                outv.at[slot], out_ref.at[pl.ds(c * chunk, chunk)],
                out_sem.at[slot])
            cp.start()
            out_copies.append(cp)

        def piece_rdma(src, dst, ssem, rsem, dev):
            return pltpu.make_async_remote_copy(
                src_ref=src, dst_ref=dst, send_sem=ssem, recv_sem=rsem,
                device_id=(dev,), device_id_type=pl.DeviceIdType.MESH,
            )

        def seed(buf, sems_s, sems_r, slot, dev):
            for p in range(P):
                r = piece_rdma(buf.at[slot, rows(p)], buf.at[0, rows(p)],
                               sems_s.at[p, 0], sems_r.at[p, 0], dev)
                r.start()
                sent.append(r)

        rsRA[4] = jnp.dot(xchunk(4), w1R,
                          preferred_element_type=jnp.float32
                          ).astype(jnp.bfloat16)
        seed(rsRA, rsRA_s, rsRA_r, 4, right)
        rsLA[3] = jnp.dot(xchunk(5), w1R,
                          preferred_element_type=jnp.float32
                          ).astype(jnp.bfloat16)
        seed(rsLA, rsLA_s, rsLA_r, 3, left)
        rsRB[3] = jnp.dot(xchunk(3), w1L,
                          preferred_element_type=jnp.float32
                          ).astype(jnp.bfloat16)
        seed(rsRB, rsRB_s, rsRB_r, 3, right)
        rsLB[4] = jnp.dot(xchunk(4), w1L,
                          preferred_element_type=jnp.float32
                          ).astype(jnp.bfloat16)
        seed(rsLB, rsLB_s, rsLB_r, 4, left)

        for o in (3, 6, 2, 5, 1, 7, 0):
            part_ref[o] = jnp.dot(xchunk(o), w1,
                                  preferred_element_type=jnp.float32
                                  ).astype(jnp.bfloat16)

        def colsA(a):
            return a[:, :hd]

        def colsB(a):
            return a[:, hd:]

        def rs_step(buf, ss, sr, s, p, o_add, half, dev, fwd):
            piece_rdma(buf.at[s, rows(p)], buf.at[s, rows(p)],
                       ss.at[p, s], sr.at[p, s], dev).wait_recv()
            padd = part_ref[o_add][p * rh:(p + 1) * rh, :]
            padd = colsA(padd) if half == "A" else colsB(padd)
            buf[s, rows(p)] = buf[s, rows(p)] + padd
            if fwd:
                r = piece_rdma(buf.at[s, rows(p)], buf.at[s + 1, rows(p)],
                               ss.at[p, s + 1], sr.at[p, s + 1], dev)
                r.start()
                sent.append(r)

        for s in (0, 1):
            for p in range(P):
                rs_step(rsRA, rsRA_s, rsRA_r, s, p, 3 - s, "A", right, True)
                rs_step(rsLA, rsLA_s, rsLA_r, s, p, 6 + s, "A", left, True)
                rs_step(rsRB, rsRB_s, rsRB_r, s, p, 2 - s, "B", right, True)
                rs_step(rsLB, rsLB_s, rsLB_r, s, p, 5 + s, "B", left, True)
        for p in range(P):
            rs_step(rsRA, rsRA_s, rsRA_r, 2, p, 1, "A", right, True)
            rs_step(rsLB, rsLB_s, rsLB_r, 2, p, 7, "B", left, True)

        for p in range(P):
            rp = rows(p)
            piece_rdma(rsRA.at[3, rp], rsRA.at[3, rp],
                       rsRA_s.at[p, 3], rsRA_r.at[p, 3], right).wait_recv()
            piece_rdma(rsLA.at[2, rp], rsLA.at[2, rp],
                       rsLA_s.at[p, 2], rsLA_r.at[p, 2], left).wait_recv()
            own_ref[rp, :hd] = (rsRA[3, rp] + rsLA[2, rp]
                                + part_ref[0][p * rh:(p + 1) * rh, :hd])
            piece_rdma(rsLB.at[3, rp], rsLB.at[3, rp],
                       rsLB_s.at[p, 3], rsLB_r.at[p, 3], left).wait_recv()
            piece_rdma(rsRB.at[2, rp], rsRB.at[2, rp],
                       rsRB_s.at[p, 2], rsRB_r.at[p, 2], right).wait_recv()
            own_ref[rp, hd:] = (rsLB[3, rp] + rsRB[2, rp]
                                + part_ref[0][p * rh:(p + 1) * rh, hd:])
            for dev, sems_s, sems_r, buf in ((right, agR_s, agR_r, agR),
                                             (left, agL_s, agL_r, agL)):
                r = piece_rdma(own_ref.at[rp], buf.at[0, rows(p)],
                               sems_s.at[p, 0], sems_r.at[p, 0], dev)
                r.start()
                sent.append(r)

        store_out(0, i, (
            jnp.dot(own_ref[:, :hd], w2R, preferred_element_type=jnp.float32)
            + jnp.dot(own_ref[:, hd:], w2L,
                      preferred_element_type=jnp.float32)
        ))

        for s in range(4):
            for p in range(P):
                for buf, ss, sr, dev, half0 in (
                        (agR, agR_s, agR_r, right, 0),
                        (agL, agL_s, agL_r, left, hd)):
                    if s == 3:
                        cs = pl.ds(half0, hd)
                        piece_rdma(buf.at[s, rows(p), cs],
                                   buf.at[s, rows(p), cs],
                                   ss.at[p, s], sr.at[p, s], dev).wait_recv()
                    else:
                        piece_rdma(buf.at[s, rows(p)], buf.at[s, rows(p)],
                                   ss.at[p, s], sr.at[p, s], dev).wait_recv()
                    if s < 2:
                        r = piece_rdma(buf.at[s, rows(p)],
                                       buf.at[s + 1, rows(p)],
                                       ss.at[p, s + 1], sr.at[p, s + 1], dev)
                        r.start()
                        sent.append(r)
                    elif s == 2:
                        cs = pl.ds(half0, hd)
                        r = piece_rdma(buf.at[s, rows(p), cs],
                                       buf.at[s + 1, rows(p), cs],
                                       ss.at[p, s + 1], sr.at[p, s + 1], dev)
                        r.start()
                        sent.append(r)
            if s < 3:
                cR = lax.rem(i - 1 - s + N_DEV, N_DEV)
                cL = lax.rem(i + 1 + s, N_DEV)
                store_out(1 + 2 * s, cR, (
                    jnp.dot(agR[s, :, :hd], w2R,
                            preferred_element_type=jnp.float32)
                    + jnp.dot(agR[s, :, hd:], w2L,
                              preferred_element_type=jnp.float32)
                ))
                store_out(2 + 2 * s, cL, (
                    jnp.dot(agL[s, :, :hd], w2R,
                            preferred_element_type=jnp.float32)
                    + jnp.dot(agL[s, :, hd:], w2L,
                              preferred_element_type=jnp.float32)
                ))
            else:
                c4 = lax.rem(i + 4, N_DEV)
                store_out(7, c4, (
                    jnp.dot(agR[3, :, :hd], w2R,
                            preferred_element_type=jnp.float32)
                    + jnp.dot(agL[3, :, hd:], w2L,
                              preferred_element_type=jnp.float32)
                ))

        for r in sent:
            r.wait_send()
        for cp in out_copies:
            cp.wait()

    return pl.pallas_call(
        body,
        out_shape=jax.ShapeDtypeStruct((m, f), jnp.float32),
        in_specs=[
            pl.BlockSpec(memory_space=pltpu.MemorySpace.HBM),
            pl.BlockSpec(memory_space=pltpu.VMEM),
            pl.BlockSpec(memory_space=pltpu.VMEM),
        ],
        out_specs=pl.BlockSpec(memory_space=pltpu.MemorySpace.HBM),
        scratch_shapes=[
            pltpu.VMEM((N_DEV, chunk, k), jnp.float32),
            pltpu.VMEM((N_DEV, chunk, d), jnp.bfloat16),
            pltpu.VMEM((5, chunk, hd), jnp.bfloat16),
            pltpu.VMEM((4, chunk, hd), jnp.bfloat16),
            pltpu.VMEM((4, chunk, hd), jnp.bfloat16),
            pltpu.VMEM((5, chunk, hd), jnp.bfloat16),
            pltpu.VMEM((chunk, d), jnp.bfloat16),
            pltpu.VMEM((4, chunk, d), jnp.bfloat16),
            pltpu.VMEM((4, chunk, d), jnp.bfloat16),
            pltpu.VMEM((N_DEV, chunk, f), jnp.float32),
            pltpu.SemaphoreType.DMA((N_DEV,)),
            pltpu.SemaphoreType.DMA((N_DEV,)),
            pltpu.SemaphoreType.DMA((P, 4)),
            pltpu.SemaphoreType.DMA((P, 4)),
            pltpu.SemaphoreType.DMA((P, 3)),
            pltpu.SemaphoreType.DMA((P, 3)),
            pltpu.SemaphoreType.DMA((P, 3)),
            pltpu.SemaphoreType.DMA((P, 3)),
            pltpu.SemaphoreType.DMA((P, 4)),
            pltpu.SemaphoreType.DMA((P, 4)),
            pltpu.SemaphoreType.DMA((P, 4)),
            pltpu.SemaphoreType.DMA((P, 4)),
            pltpu.SemaphoreType.DMA((P, 4)),
            pltpu.SemaphoreType.DMA((P, 4)),
        ],
        compiler_params=pltpu.CompilerParams(collective_id=0),
    )(x, W1, W2)


# device time: 56003 ns/iter; 1.0194x vs baseline; 1.0194x over previous
import jax
import jax.numpy as jnp
from jax import lax
from jax.experimental import pallas as pl
from jax.experimental.pallas import tpu as pltpu

N_DEV = 8
P = 2


def kernel(x, W1, W2):
    m, k = x.shape
    _, d = W1.shape
    _, f = W2.shape
    chunk = m // N_DEV
    hd = d // 2
    rh = chunk // P

    def body(x_ref, w1_ref, w2_ref, out_ref, part_ref,
             rsRA, rsLA, rsRB, rsLB, own_ref, agR, agL,
             rsRA_s, rsRA_r, rsLA_s, rsLA_r, rsRB_s, rsRB_r,
             rsLB_s, rsLB_r, agR_s, agR_r, agL_s, agL_r):
        i = lax.axis_index("i")
        left = lax.rem(i - 1 + N_DEV, N_DEV)
        right = lax.rem(i + 1, N_DEV)

        barrier_sem = pltpu.get_barrier_semaphore()
        for nbr in (left, right):
            pl.semaphore_signal(
                barrier_sem, inc=1,
                device_id=(nbr,), device_id_type=pl.DeviceIdType.MESH,
            )
        pl.semaphore_wait(barrier_sem, 2)

        w1 = w1_ref[...].astype(jnp.bfloat16)
        w1R = w1[:, :hd]
        w1L = w1[:, hd:]
        w2R = w2_ref[:hd, :].astype(jnp.bfloat16)
        w2L = w2_ref[hd:, :].astype(jnp.bfloat16)

        def xchunk(o):
            c = lax.rem(i + o, N_DEV)
            return x_ref[pl.ds(c * chunk, chunk), :].astype(jnp.bfloat16)

        def rows(p):
            return pl.ds(p * rh, rh)

        sent = []

        def piece_rdma(src, dst, ssem, rsem, dev):
            r = pltpu.make_async_remote_copy(
                src_ref=src, dst_ref=dst, send_sem=ssem, recv_sem=rsem,
                device_id=(dev,), device_id_type=pl.DeviceIdType.MESH,
            )
            return r

        def seed(buf, sems_s, sems_r, slot, dev):
            for p in range(P):
                r = piece_rdma(buf.at[slot, rows(p)], buf.at[0, rows(p)],
                               sems_s.at[p, 0], sems_r.at[p, 0], dev)
                r.start()
                sent.append(r)

        rsRA[4] = jnp.dot(xchunk(4), w1R,
                          preferred_element_type=jnp.float32
                          ).astype(jnp.bfloat16)
        seed(rsRA, rsRA_s, rsRA_r, 4, right)
        rsLA[3] = jnp.dot(xchunk(5), w1R,
                          preferred_element_type=jnp.float32
                          ).astype(jnp.bfloat16)
        seed(rsLA, rsLA_s, rsLA_r, 3, left)
        rsRB[3] = jnp.dot(xchunk(3), w1L,
                          preferred_element_type=jnp.float32
                          ).astype(jnp.bfloat16)
        seed(rsRB, rsRB_s, rsRB_r, 3, right)
        rsLB[4] = jnp.dot(xchunk(4), w1L,
                          preferred_element_type=jnp.float32
                          ).astype(jnp.bfloat16)
        seed(rsLB, rsLB_s, rsLB_r, 4, left)

        for o in (3, 6, 2, 5, 1, 7, 0):
            part_ref[o] = jnp.dot(xchunk(o), w1,
                                  preferred_element_type=jnp.float32
                                  ).astype(jnp.bfloat16)

        def colsA(a):
            return a[:, :hd]

        def colsB(a):
            return a[:, hd:]

        def rs_step(buf, ss, sr, s, p, o_add, half, dev, fwd):
            piece_rdma(buf.at[s, rows(p)], buf.at[s, rows(p)],
                       ss.at[p, s], sr.at[p, s], dev).wait_recv()
            padd = part_ref[o_add][p * rh:(p + 1) * rh, :]
            padd = colsA(padd) if half == "A" else colsB(padd)
            buf[s, rows(p)] = buf[s, rows(p)] + padd
            if fwd:
                r = piece_rdma(buf.at[s, rows(p)], buf.at[s + 1, rows(p)],
                               ss.at[p, s + 1], sr.at[p, s + 1], dev)
                r.start()
                sent.append(r)

        for s in (0, 1):
            for p in range(P):
                rs_step(rsRA, rsRA_s, rsRA_r, s, p, 3 - s, "A", right, True)
                rs_step(rsLA, rsLA_s, rsLA_r, s, p, 6 + s, "A", left, True)
                rs_step(rsRB, rsRB_s, rsRB_r, s, p, 2 - s, "B", right, True)
                rs_step(rsLB, rsLB_s, rsLB_r, s, p, 5 + s, "B", left, True)
        for p in range(P):
            rs_step(rsRA, rsRA_s, rsRA_r, 2, p, 1, "A", right, True)
            rs_step(rsLB, rsLB_s, rsLB_r, 2, p, 7, "B", left, True)

        for p in range(P):
            rp = rows(p)
            piece_rdma(rsRA.at[3, rp], rsRA.at[3, rp],
                       rsRA_s.at[p, 3], rsRA_r.at[p, 3], right).wait_recv()
            piece_rdma(rsLA.at[2, rp], rsLA.at[2, rp],
                       rsLA_s.at[p, 2], rsLA_r.at[p, 2], left).wait_recv()
            own_ref[rp, :hd] = (rsRA[3, rp] + rsLA[2, rp]
                                + part_ref[0][p * rh:(p + 1) * rh, :hd])
            piece_rdma(rsLB.at[3, rp], rsLB.at[3, rp],
                       rsLB_s.at[p, 3], rsLB_r.at[p, 3], left).wait_recv()
            piece_rdma(rsRB.at[2, rp], rsRB.at[2, rp],
                       rsRB_s.at[p, 2], rsRB_r.at[p, 2], right).wait_recv()
            own_ref[rp, hd:] = (rsLB[3, rp] + rsRB[2, rp]
                                + part_ref[0][p * rh:(p + 1) * rh, hd:])
            for dev, sems_s, sems_r, buf in ((right, agR_s, agR_r, agR),
                                             (left, agL_s, agL_r, agL)):
                r = piece_rdma(own_ref.at[rp], buf.at[0, rows(p)],
                               sems_s.at[p, 0], sems_r.at[p, 0], dev)
                r.start()
                sent.append(r)

        out_ref[pl.ds(i * chunk, chunk), :] = (
            jnp.dot(own_ref[:, :hd], w2R, preferred_element_type=jnp.float32)
            + jnp.dot(own_ref[:, hd:], w2L, preferred_element_type=jnp.float32)
        )

        for s in range(4):
            for p in range(P):
                for buf, ss, sr, dev, half0 in (
                        (agR, agR_s, agR_r, right, 0),
                        (agL, agL_s, agL_r, left, hd)):
                    if s == 3:
                        cs = pl.ds(half0, hd)
                        piece_rdma(buf.at[s, rows(p), cs],
                                   buf.at[s, rows(p), cs],
                                   ss.at[p, s], sr.at[p, s], dev).wait_recv()
                    else:
                        piece_rdma(buf.at[s, rows(p)], buf.at[s, rows(p)],
                                   ss.at[p, s], sr.at[p, s], dev).wait_recv()
                    if s < 2:
                        r = piece_rdma(buf.at[s, rows(p)],
                                       buf.at[s + 1, rows(p)],
                                       ss.at[p, s + 1], sr.at[p, s + 1], dev)
                        r.start()
                        sent.append(r)
                    elif s == 2:
                        cs = pl.ds(half0, hd)
                        r = piece_rdma(buf.at[s, rows(p), cs],
                                       buf.at[s + 1, rows(p), cs],
                                       ss.at[p, s + 1], sr.at[p, s + 1], dev)
                        r.start()
                        sent.append(r)
            if s < 3:
                cR = lax.rem(i - 1 - s + N_DEV, N_DEV)
                cL = lax.rem(i + 1 + s, N_DEV)
                out_ref[pl.ds(cR * chunk, chunk), :] = (
                    jnp.dot(agR[s, :, :hd], w2R,
                            preferred_element_type=jnp.float32)
                    + jnp.dot(agR[s, :, hd:], w2L,
                              preferred_element_type=jnp.float32)
                )
                out_ref[pl.ds(cL * chunk, chunk), :] = (
                    jnp.dot(agL[s, :, :hd], w2R,
                            preferred_element_type=jnp.float32)
                    + jnp.dot(agL[s, :, hd:], w2L,
                              preferred_element_type=jnp.float32)
                )
            else:
                c4 = lax.rem(i + 4, N_DEV)
                out_ref[pl.ds(c4 * chunk, chunk), :] = (
                    jnp.dot(agR[3, :, :hd], w2R,
                            preferred_element_type=jnp.float32)
                    + jnp.dot(agL[3, :, hd:], w2L,
                              preferred_element_type=jnp.float32)
                )

        for r in sent:
            r.wait_send()

    return pl.pallas_call(
        body,
        out_shape=jax.ShapeDtypeStruct((m, f), jnp.float32),
        in_specs=[
            pl.BlockSpec(memory_space=pltpu.VMEM),
            pl.BlockSpec(memory_space=pltpu.VMEM),
            pl.BlockSpec(memory_space=pltpu.VMEM),
        ],
        out_specs=pl.BlockSpec(memory_space=pltpu.VMEM),
        scratch_shapes=[
            pltpu.VMEM((N_DEV, chunk, d), jnp.bfloat16),
            pltpu.VMEM((5, chunk, hd), jnp.bfloat16),
            pltpu.VMEM((4, chunk, hd), jnp.bfloat16),
            pltpu.VMEM((4, chunk, hd), jnp.bfloat16),
            pltpu.VMEM((5, chunk, hd), jnp.bfloat16),
            pltpu.VMEM((chunk, d), jnp.bfloat16),
            pltpu.VMEM((4, chunk, d), jnp.bfloat16),
            pltpu.VMEM((4, chunk, d), jnp.bfloat16),
            pltpu.SemaphoreType.DMA((P, 4)),
            pltpu.SemaphoreType.DMA((P, 4)),
            pltpu.SemaphoreType.DMA((P, 3)),
            pltpu.SemaphoreType.DMA((P, 3)),
            pltpu.SemaphoreType.DMA((P, 3)),
            pltpu.SemaphoreType.DMA((P, 3)),
            pltpu.SemaphoreType.DMA((P, 4)),
            pltpu.SemaphoreType.DMA((P, 4)),
            pltpu.SemaphoreType.DMA((P, 4)),
            pltpu.SemaphoreType.DMA((P, 4)),
            pltpu.SemaphoreType.DMA((P, 4)),
            pltpu.SemaphoreType.DMA((P, 4)),
        ],
        compiler_params=pltpu.CompilerParams(collective_id=0),
    )(x, W1, W2)
